# Initial kernel scaffold; baseline (speedup 1.0000x reference)
import jax
import jax.numpy as jnp
from jax.experimental import pallas as pl


def kernel(x, edge_index, W1, b1, W2, b2, W3, b3, ln_g, ln_b, gn_g, gn_b):
    raise NotImplementedError("write your pallas kernel here")


# speedup vs baseline: 6.2136x; 6.2136x over previous
import functools

import jax
import jax.numpy as jnp
from jax import lax
from jax.experimental import pallas as pl
from jax.experimental.pallas import tpu as pltpu
from jax.experimental.pallas import tpu_sc as plsc

_N = 10000
_E = 320000
_D = 128

_NW = 32
_CH = 128
_NCH = 80
_EPT = _CH * _NCH
_NP = 10240
_RPT = _NP // 16
_R = 512

_mesh = plsc.VectorSubcoreMesh(core_axis_name="c", subcore_axis_name="s")


def _deg_body(dst_hbm, const_hbm, out_hbm, dst_v, ones_v, zer_v, accum, sem):
    cid = lax.axis_index("c")
    sid = lax.axis_index("s")
    wid = cid * 16 + sid
    pltpu.sync_copy(dst_hbm.at[wid], dst_v)
    pltpu.sync_copy(const_hbm.at[0], zer_v)
    pltpu.sync_copy(const_hbm.at[1], ones_v)
    base = sid * _RPT
    for i in range(_RPT // _CH):
        pltpu.sync_copy(zer_v, accum.at[pl.ds(base + i * _CH, _CH)])
    plsc.subcore_barrier()

    def chunk(j, carry):
        pltpu.sync_copy(ones_v, accum.at[dst_v.at[j]], add=True)
        return carry

    lax.fori_loop(0, _NCH, chunk, 0)
    plsc.subcore_barrier()
    pltpu.sync_copy(accum.at[pl.ds(base, _RPT)], out_hbm.at[cid, pl.ds(base, _RPT)])


_deg_call = functools.partial(
    pl.kernel,
    out_type=jax.ShapeDtypeStruct((2, _NP, 16), jnp.float32),
    mesh=_mesh,
    scratch_types=[
        pltpu.VMEM((_NCH, _CH), jnp.int32),
        pltpu.VMEM((_CH, 16), jnp.float32),
        pltpu.VMEM((_CH, 16), jnp.float32),
        pltpu.VMEM_SHARED((_NP, 16), jnp.float32),
        pltpu.SemaphoreType.DMA,
    ],
)(_deg_body)


_DH = _D // 2


def _gs_body(y_hbm, srcA_hbm, srcB_hbm, dst_hbm, zer_hbm, out_hbm,
             srcA_v, srcB_v, dst_v, zer_v, buf0, buf1, accum, sem0, sem1):
    cid = lax.axis_index("c")
    sid = lax.axis_index("s")
    wid = cid * 16 + sid
    pltpu.sync_copy(srcA_hbm.at[wid], srcA_v)
    pltpu.sync_copy(srcB_hbm.at[wid], srcB_v)
    pltpu.sync_copy(dst_hbm.at[wid], dst_v)
    pltpu.sync_copy(zer_hbm, zer_v)
    base = sid * _RPT
    for half, idx_v in ((0, srcA_v), (1, srcB_v)):
        for i in range(_RPT // _CH):
            pltpu.sync_copy(zer_v, accum.at[pl.ds(base + i * _CH, _CH)])
        plsc.subcore_barrier()

        def pair(p, carry, idx_v=idx_v):
            j0 = p * 2
            j1 = j0 + 1
            c0 = pltpu.async_copy(y_hbm.at[idx_v.at[j0]], buf0, sem0)
            c1 = pltpu.async_copy(y_hbm.at[idx_v.at[j1]], buf1, sem1)
            c0.wait()
            pltpu.sync_copy(buf0, accum.at[dst_v.at[j0]], add=True)
            c1.wait()
            pltpu.sync_copy(buf1, accum.at[dst_v.at[j1]], add=True)
            return carry

        lax.fori_loop(0, _NCH // 2, pair, 0)
        plsc.subcore_barrier()
        pltpu.sync_copy(accum.at[pl.ds(base, _RPT)],
                        out_hbm.at[cid, half, pl.ds(base, _RPT)])


_gs_call = functools.partial(
    pl.kernel,
    out_type=jax.ShapeDtypeStruct((2, 2, _NP, _DH), jnp.float32),
    mesh=_mesh,
    compiler_params=pltpu.CompilerParams(use_tc_tiling_on_sc=False),
    scratch_types=[
        pltpu.VMEM((_NCH, _CH), jnp.int32),
        pltpu.VMEM((_NCH, _CH), jnp.int32),
        pltpu.VMEM((_NCH, _CH), jnp.int32),
        pltpu.VMEM((_CH, _DH), jnp.float32),
        pltpu.VMEM((_CH, _DH), jnp.float32),
        pltpu.VMEM((_CH, _DH), jnp.float32),
        pltpu.VMEM_SHARED((_NP, _DH), jnp.float32),
        pltpu.SemaphoreType.DMA,
        pltpu.SemaphoreType.DMA,
    ],
)(_gs_body)


_SQRT2 = 1.4142135623730951


def _pre_body(deg_ref, x_ref, w_ref, y_ref, dis_ref):
    deg = deg_ref[0] + deg_ref[1] + 1.0
    dis16 = lax.rsqrt(deg)
    dis = dis16[:, :1]
    xw = jnp.dot(x_ref[...], w_ref[...], preferred_element_type=jnp.float32)
    y_ref[...] = dis * xw
    dis_ref[...] = dis16


def _norm_gelu(s_ref, y_ref, dis_ref, b_ref, g_ref, bb_ref):
    dis = dis_ref[:, :1]
    s = jnp.concatenate([s_ref[0, 0] + s_ref[1, 0],
                         s_ref[0, 1] + s_ref[1, 1]], axis=-1)
    h = dis * (s + y_ref[...]) + b_ref[...]
    mu = jnp.mean(h, axis=-1, keepdims=True)
    hc = h - mu
    var = jnp.mean(hc * hc, axis=-1, keepdims=True)
    hn = hc * lax.rsqrt(var + 1e-5) * g_ref[...] + bb_ref[...]
    return 0.5 * hn * (1.0 + lax.erf(hn / _SQRT2)), dis


def _mid_body(s_ref, y_ref, dis_ref, b_ref, g_ref, bb_ref, w_ref, out_ref):
    ge, dis = _norm_gelu(s_ref, y_ref, dis_ref, b_ref, g_ref, bb_ref)
    out_ref[...] = dis * jnp.dot(ge, w_ref[...], preferred_element_type=jnp.float32)


def _fin_body(s_ref, y_ref, dis_ref, b_ref, g_ref, bb_ref, out_ref):
    ge, _ = _norm_gelu(s_ref, y_ref, dis_ref, b_ref, g_ref, bb_ref)
    out_ref[...] = ge


def _tc_pre(deg2, x_p, w):
    return pl.pallas_call(
        _pre_body,
        grid=(_NP // _R,),
        in_specs=[
            pl.BlockSpec((2, _R, 16), lambda i: (0, i, 0)),
            pl.BlockSpec((_R, _D), lambda i: (i, 0)),
            pl.BlockSpec((_D, _D), lambda i: (0, 0)),
        ],
        out_specs=[
            pl.BlockSpec((_R, _D), lambda i: (i, 0)),
            pl.BlockSpec((_R, 16), lambda i: (i, 0)),
        ],
        out_shape=[
            jax.ShapeDtypeStruct((_NP, _D), jnp.float32),
            jax.ShapeDtypeStruct((_NP, 16), jnp.float32),
        ],
    )(deg2, x_p, w)


def _row_specs():
    return [
        pl.BlockSpec((2, 2, _R, _DH), lambda i: (0, 0, i, 0)),
        pl.BlockSpec((_R, _D), lambda i: (i, 0)),
        pl.BlockSpec((_R, 16), lambda i: (i, 0)),
        pl.BlockSpec((1, _D), lambda i: (0, 0)),
        pl.BlockSpec((1, _D), lambda i: (0, 0)),
        pl.BlockSpec((1, _D), lambda i: (0, 0)),
    ]


def _tc_mid(s2, y, dis, b, g, bb, w):
    return pl.pallas_call(
        _mid_body,
        grid=(_NP // _R,),
        in_specs=_row_specs() + [pl.BlockSpec((_D, _D), lambda i: (0, 0))],
        out_specs=pl.BlockSpec((_R, _D), lambda i: (i, 0)),
        out_shape=jax.ShapeDtypeStruct((_NP, _D), jnp.float32),
    )(s2, y, dis, b, g, bb, w)


def _tc_fin(s2, y, dis, b, g, bb):
    return pl.pallas_call(
        _fin_body,
        grid=(_NP // _R,),
        in_specs=_row_specs(),
        out_specs=pl.BlockSpec((_R, _D), lambda i: (i, 0)),
        out_shape=jax.ShapeDtypeStruct((_NP, _D), jnp.float32),
    )(s2, y, dis, b, g, bb)


def kernel(x, edge_index, W1, b1, W2, b2, W3, b3, ln_g, ln_b, gn_g, gn_b):
    pad = _NW * _EPT - _E
    src = jnp.concatenate([edge_index[0], jnp.zeros((pad,), jnp.int32)])
    dst = jnp.concatenate([edge_index[1], jnp.full((pad,), _NP - 1, jnp.int32)])
    srcA = (src * 2).reshape(_NW, _NCH, _CH)
    srcB = (src * 2 + 1).reshape(_NW, _NCH, _CH)
    dst = dst.reshape(_NW, _NCH, _CH)

    const16 = jnp.stack([jnp.zeros((_CH, 16), jnp.float32),
                         jnp.ones((_CH, 16), jnp.float32)])
    zer_d = jnp.zeros((_CH, _DH), jnp.float32)
    x_p = jnp.concatenate([x, jnp.zeros((_NP - _N, _D), jnp.float32)])

    b1r = b1.reshape(1, _D)
    b2r = b2.reshape(1, _D)
    b3r = b3.reshape(1, _D)
    ln_gr = ln_g.reshape(1, _D)
    ln_br = ln_b.reshape(1, _D)
    gn_gr = gn_g.reshape(1, _D)
    gn_br = gn_b.reshape(1, _D)

    deg2 = _deg_call(dst, const16)
    y1, dis = _tc_pre(deg2, x_p, W1)
    s1 = _gs_call(y1.reshape(2 * _NP, _DH), srcA, srcB, dst, zer_d)
    y2 = _tc_mid(s1, y1, dis, b1r, ln_gr, ln_br, W2)
    s2 = _gs_call(y2.reshape(2 * _NP, _DH), srcA, srcB, dst, zer_d)
    y3 = _tc_mid(s2, y2, dis, b2r, gn_gr, gn_br, W3)
    s3 = _gs_call(y3.reshape(2 * _NP, _DH), srcA, srcB, dst, zer_d)
    h = _tc_fin(s3, y3, dis, b3r, ln_gr, ln_br)
    return h[:_N]
